# baseline (speedup 1.0000x reference)
import jax
import jax.numpy as jnp
from jax.experimental import pallas as pl
from jax.experimental.pallas import tpu as pltpu
from jax.experimental.pallas import tpu_sc as plsc

EMBED_DIM = 64
SCALE = 8.0
WINDOW = 128
LANES = 16


def _sc_gather_scale(table, idx, num_indices):
    mesh = plsc.VectorSubcoreMesh(core_axis_name="c", subcore_axis_name="s")

    @pl.kernel(
        out_type=jax.ShapeDtypeStruct((num_indices, EMBED_DIM), table.dtype),
        mesh=mesh,
        compiler_params=pltpu.CompilerParams(use_tc_tiling_on_sc=False),
    )
    def k(table_hbm, i_hbm, o_hbm):
        def body(i_vmem, o_vmem):
            pltpu.sync_copy(table_hbm.at[i_vmem.at[0]], o_vmem)

            @pl.loop(0, WINDOW)
            def _(r):
                for c in range(0, EMBED_DIM, LANES):
                    o_vmem.at[r, pl.ds(c, LANES)][...] = (
                        o_vmem.at[r, pl.ds(c, LANES)][...] * SCALE
                    )

        pltpu.emit_pipeline(
            body,
            grid=(num_indices // WINDOW,),
            in_specs=[pl.BlockSpec((1, WINDOW), index_map=lambda i: (0, i))],
            out_specs=[
                pl.BlockSpec((WINDOW, EMBED_DIM), index_map=lambda i: (i, 0))
            ],
            core_axis_name=("c", "s"),
            dimension_semantics=(pltpu.PARALLEL,),
        )(i_hbm, o_hbm)

    return k(table, idx)


def kernel(x, table):
    b, h = x.shape
    n = b * h
    idx = x.reshape(1, n).astype(jnp.int32)
    out = _sc_gather_scale(table, idx, n)
    return out.reshape(b, h, EMBED_DIM)


# speedup vs baseline: 1.4705x; 1.4705x over previous
import jax
import jax.numpy as jnp
from jax.experimental import pallas as pl
from jax.experimental.pallas import tpu as pltpu
from jax.experimental.pallas import tpu_sc as plsc

EMBED_DIM = 64
HIST = 50
SCALE = 8.0
LANES = 16

NC, NS = 2, 16
NW = NC * NS
IPR = 2 * HIST
ROWS_PER_CHUNK = 2
XPC = ROWS_PER_CHUNK * 2
NBUF = 2


def _scale_into(out3, src, b_base):
    for s in range(XPC):
        @pl.loop(0, HIST)
        def _(rr):
            for c in range(0, EMBED_DIM, LANES):
                out3.at[s, rr, pl.ds(c, LANES)][...] = (
                    src.at[s * HIST + rr, pl.ds(c, LANES)][...] * SCALE
                )


def _sc_gather_scale(table, idx, batch):
    num_rows = idx.shape[0]
    chunks = num_rows // ROWS_PER_CHUNK
    cpw = chunks // NW

    mesh = plsc.VectorSubcoreMesh(core_axis_name="c", subcore_axis_name="s")

    @pl.kernel(
        out_type=jax.ShapeDtypeStruct((batch, HIST, EMBED_DIM), table.dtype),
        mesh=mesh,
        scratch_types=[
            pltpu.VMEM((num_rows // NW, IPR), jnp.int32),
            pltpu.VMEM((NBUF, IPR * ROWS_PER_CHUNK, EMBED_DIM), jnp.float32),
            pltpu.VMEM((NBUF, XPC, HIST, EMBED_DIM), jnp.float32),
            pltpu.SemaphoreType.DMA((NBUF,)),
            pltpu.SemaphoreType.DMA((NBUF,)),
        ],
        compiler_params=pltpu.CompilerParams(use_tc_tiling_on_sc=False),
    )
    def k(table_hbm, i_hbm, o_hbm, idx_v, in_v, out_v, gsem, wsem):
        wid = jax.lax.axis_index("s") * NC + jax.lax.axis_index("c")
        row0 = wid * (num_rows // NW)
        chunk0 = wid * cpw

        pltpu.sync_copy(i_hbm.at[pl.ds(row0, num_rows // NW)], idx_v)

        def start_gather(cc, b):
            for j in range(ROWS_PER_CHUNK):
                pltpu.async_copy(
                    table_hbm.at[idx_v.at[cc * ROWS_PER_CHUNK + j]],
                    in_v.at[b, pl.ds(j * IPR, IPR)],
                    gsem.at[b],
                )

        def wait_gather(cc, b):
            for j in range(ROWS_PER_CHUNK):
                pltpu.make_async_copy(
                    table_hbm.at[idx_v.at[cc * ROWS_PER_CHUNK + j]],
                    in_v.at[b, pl.ds(j * IPR, IPR)],
                    gsem.at[b],
                ).wait()

        def write_dst(cc):
            return o_hbm.at[pl.ds((chunk0 + cc) * XPC, XPC)]

        for b in range(NBUF):
            start_gather(b, b)
        for b in range(NBUF):
            wait_gather(b, b)
            _scale_into(out_v.at[b], in_v.at[b], b)
            pltpu.async_copy(out_v.at[b], write_dst(b), wsem.at[b])
            start_gather(NBUF + b, b)

        @pl.loop(1, cpw // NBUF)
        def _(r):
            for b in range(NBUF):
                cc = r * NBUF + b
                wait_gather(cc, b)
                pltpu.make_async_copy(
                    out_v.at[b], write_dst(cc - NBUF), wsem.at[b]
                ).wait()
                _scale_into(out_v.at[b], in_v.at[b], b)
                pltpu.async_copy(out_v.at[b], write_dst(cc), wsem.at[b])

                @pl.when(cc + NBUF < cpw)
                def _():
                    start_gather(cc + NBUF, b)

        for b in range(NBUF):
            pltpu.make_async_copy(
                out_v.at[b], write_dst(cpw - NBUF + b), wsem.at[b]
            ).wait()

    return k(table, idx)


def kernel(x, table):
    b, h = x.shape
    idx = x.astype(jnp.int32).reshape(b * h // IPR, IPR)
    return _sc_gather_scale(table, idx, b)
